# baseline (speedup 1.0000x reference)
import jax
import jax.numpy as jnp
from jax import lax
from jax.experimental import pallas as pl
from jax.experimental.pallas import tpu as pltpu
from jax.experimental.pallas import tpu_sc as plsc

N = 10000
E = 320000
H = 128
C_OUT = 10
NC = 2
NS = 16
NW = NC * NS
RPT = N // NS
T = 128
DT = 80
DCH = E // (NS * DT)
SCH = 80
EPW = SCH * T
NPAD = T
DEG_W = 16

_mesh = plsc.VectorSubcoreMesh(core_axis_name="c", subcore_axis_name="s")


DK = 10
DGRP = DCH // DK


def _deg_body(ei_ref, ones_ref, zeros_ref, out_ref, idx_v, ones_v, acc_sh,
              d0, d1, d2, d3, d4, d5, d6, d7, d8, d9):
    c = lax.axis_index("c")
    s = lax.axis_index("s")
    dsem = [d0, d1, d2, d3, d4, d5, d6, d7, d8, d9]

    @pl.when(s == 0)
    def _():
        pltpu.sync_copy(zeros_ref, acc_sh)

    pltpu.sync_copy(ei_ref.at[c, s], idx_v)
    pltpu.sync_copy(ones_ref, ones_v)
    plsc.subcore_barrier()

    def body(j, carry):
        descs = []
        for b in range(DK):
            descs.append(pltpu.async_copy(
                ones_v, acc_sh.at[idx_v.at[j * DK + b]], dsem[b], add=True))
        for d in descs:
            d.wait()
        return carry

    lax.fori_loop(0, DGRP, body, 0)
    plsc.subcore_barrier()

    @pl.when(s == 0)
    def _():
        pltpu.sync_copy(acc_sh, out_ref.at[c])


_deg_kernel = pl.kernel(
    _deg_body,
    out_type=jax.ShapeDtypeStruct((NC, N), jnp.float32),
    mesh=_mesh,
    scratch_types=[
        pltpu.VMEM((DCH, DT), jnp.int32),
        pltpu.VMEM((DT,), jnp.float32),
        pltpu.VMEM_SHARED((N,), jnp.float32),
        pltpu.SemaphoreType.DMA,
        pltpu.SemaphoreType.DMA,
        pltpu.SemaphoreType.DMA,
        pltpu.SemaphoreType.DMA,
        pltpu.SemaphoreType.DMA,
        pltpu.SemaphoreType.DMA,
        pltpu.SemaphoreType.DMA,
        pltpu.SemaphoreType.DMA,
        pltpu.SemaphoreType.DMA,
        pltpu.SemaphoreType.DMA,
    ],
    compiler_params=pltpu.CompilerParams(use_tc_tiling_on_sc=False),
)


NBUF = 2
SGRP = SCH // NBUF
SLOTS = 4
SGJ = SGRP // SLOTS


def _agg_body(hw_ref, ei_ref, zeros_ref, out_ref,
              ss0, ss1, ss2, ss3, ds0, ds1, ds2, ds3,
              r0, r1, acc_sh,
              l0, l1, l2, l3, g0, g1, s0, s1):
    c = lax.axis_index("c")
    s = lax.axis_index("s")
    wid = s * NC + c
    ss = [ss0, ss1, ss2, ss3]
    ds = [ds0, ds1, ds2, ds3]
    rows = [r0, r1]
    lsem = [l0, l1, l2, l3]
    gsem = [g0, g1]
    ssem = [s0, s1]

    def slab_src(g):
        return ei_ref.at[0, wid, pl.ds(g * NBUF, NBUF)]

    def slab_dst(g):
        return ei_ref.at[1, wid, pl.ds(g * NBUF, NBUF)]

    pltpu.sync_copy(zeros_ref, acc_sh.at[pl.ds(s * RPT, RPT)])
    pltpu.sync_copy(slab_src(0), ss[0])
    pltpu.sync_copy(slab_dst(0), ds[0])
    for k in range(1, SLOTS):
        pltpu.async_copy(slab_src(k), ss[k], lsem[k])
        pltpu.async_copy(slab_dst(k), ds[k], lsem[k])
    plsc.subcore_barrier()

    for b in range(NBUF):
        pltpu.async_copy(hw_ref.at[ss[0].at[b]], rows[b], gsem[b])

    def run_group(g, kk, knext, last):
        if not last:
            pltpu.make_async_copy(slab_src(g), ss[knext], lsem[knext]).wait()
            pltpu.make_async_copy(slab_dst(g), ds[knext], lsem[knext]).wait()
        descs = [None] * NBUF
        for b in range(NBUF):
            pltpu.make_async_copy(hw_ref.at[ss[kk].at[b]], rows[b],
                                  gsem[b]).wait()
            if b > 0:
                descs[b - 1].wait()
                if not last:
                    pltpu.async_copy(hw_ref.at[ss[knext].at[b - 1]],
                                     rows[b - 1], gsem[b - 1])
            descs[b] = pltpu.async_copy(rows[b], acc_sh.at[ds[kk].at[b]],
                                        ssem[b], add=True)
        descs[NBUF - 1].wait()
        if not last:
            pltpu.async_copy(hw_ref.at[ss[knext].at[NBUF - 1]],
                             rows[NBUF - 1], gsem[NBUF - 1])

    def body(j, carry):
        for gg in range(SLOTS):
            g = j * SLOTS + gg
            run_group(g, gg, (gg + 1) % SLOTS, last=False)
            pltpu.async_copy(slab_src(g + SLOTS), ss[gg], lsem[gg])
            pltpu.async_copy(slab_dst(g + SLOTS), ds[gg], lsem[gg])
        return carry

    lax.fori_loop(0, SGJ - 1, body, 0)

    for gg in range(SLOTS):
        g = (SGJ - 1) * SLOTS + gg
        run_group(g, gg, (gg + 1) % SLOTS, last=(gg == SLOTS - 1))

    plsc.subcore_barrier()
    pltpu.sync_copy(acc_sh.at[pl.ds(s * RPT, RPT)], out_ref.at[c, s])


_agg_kernel = pl.kernel(
    _agg_body,
    out_type=jax.ShapeDtypeStruct((NC, NS, RPT, H), jnp.float32),
    mesh=_mesh,
    scratch_types=[
        pltpu.VMEM((NBUF, T), jnp.int32),
        pltpu.VMEM((NBUF, T), jnp.int32),
        pltpu.VMEM((NBUF, T), jnp.int32),
        pltpu.VMEM((NBUF, T), jnp.int32),
        pltpu.VMEM((NBUF, T), jnp.int32),
        pltpu.VMEM((NBUF, T), jnp.int32),
        pltpu.VMEM((NBUF, T), jnp.int32),
        pltpu.VMEM((NBUF, T), jnp.int32),
        pltpu.VMEM((T, H), jnp.float32),
        pltpu.VMEM((T, H), jnp.float32),
        pltpu.VMEM_SHARED((N + NPAD, H), jnp.float32),
        pltpu.SemaphoreType.DMA,
        pltpu.SemaphoreType.DMA,
        pltpu.SemaphoreType.DMA,
        pltpu.SemaphoreType.DMA,
        pltpu.SemaphoreType.DMA,
        pltpu.SemaphoreType.DMA,
        pltpu.SemaphoreType.DMA,
        pltpu.SemaphoreType.DMA,
    ],
    compiler_params=pltpu.CompilerParams(use_tc_tiling_on_sc=False),
)


def _ln(x, g, b):
    mu = jnp.mean(x, axis=-1, keepdims=True)
    var = jnp.mean((x - mu) ** 2, axis=-1, keepdims=True)
    return (x - mu) * lax.rsqrt(var + 1e-5) * g + b


def _gelu(x):
    return 0.5 * x * (1.0 + lax.erf(x * 0.7071067811865476))


def _inv_sqrt_deg(d):
    return lax.rsqrt(jnp.maximum(d, 1.0))


def _tc_pre_body(x_ref, w_ref, od_ref, out_ref):
    h = x_ref[...] * _inv_sqrt_deg(od_ref[...])
    out_ref[...] = jnp.dot(h, w_ref[...], preferred_element_type=jnp.float32, precision=lax.Precision.HIGHEST)


def _tc_mid_body(p_ref, id_ref, od_ref, b1_ref, g1_ref, be1_ref, w2_ref, out_ref):
    agg = p_ref[0] + p_ref[1]
    agg = agg * _inv_sqrt_deg(id_ref[...]) + b1_ref[...]
    h = _gelu(_ln(agg, g1_ref[...], be1_ref[...]))
    h = h * _inv_sqrt_deg(od_ref[...])
    out_ref[...] = jnp.dot(h, w2_ref[...], preferred_element_type=jnp.float32, precision=lax.Precision.HIGHEST)


def _tc_fin_body(p_ref, id_ref, b2_ref, g2_ref, be2_ref, wc1_ref, bc1_ref,
                 g3_ref, be3_ref, wc3_ref, bc3_ref, out_ref):
    agg = p_ref[0] + p_ref[1]
    agg = agg * _inv_sqrt_deg(id_ref[...]) + b2_ref[...]
    h = _gelu(_ln(agg, g2_ref[...], be2_ref[...]))
    hg = jnp.mean(h, axis=0, keepdims=True)
    z = jnp.dot(hg, wc1_ref[...], preferred_element_type=jnp.float32, precision=lax.Precision.HIGHEST) + bc1_ref[...]
    z = jnp.maximum(_ln(z, g3_ref[...], be3_ref[...]), 0.0)
    out_ref[...] = jnp.dot(z, wc3_ref[...], preferred_element_type=jnp.float32, precision=lax.Precision.HIGHEST) + bc3_ref[...]


def kernel(x, edge_index, W1, b1, W2, b2, g1, be1, g2, be2, g3, be3,
           Wc1, bc1, Wc3, bc3):
    ei_deg = edge_index.reshape(2, NS, DCH, DT)
    pad = EPW - E // NW
    apad = jnp.arange(pad, dtype=jnp.int32)
    src_pad = jnp.broadcast_to((apad * 131) % N, (NW, pad))
    dst_pad = jnp.broadcast_to(N + (apad % NPAD), (NW, pad))
    ei_sc = jnp.concatenate(
        [edge_index.reshape(2, NW, E // NW),
         jnp.stack([src_pad, dst_pad])], axis=-1).reshape(2, NW, SCH, T)
    ones_deg = jnp.ones((DT,), jnp.float32)
    zeros_deg = jnp.zeros((N,), jnp.float32)
    zeros_feat = jnp.zeros((RPT, H), jnp.float32)

    deg = _deg_kernel(ei_deg, ones_deg, zeros_deg)
    od = deg[0].reshape(N, 1)
    idg = deg[1].reshape(N, 1)

    hw1 = pl.pallas_call(
        _tc_pre_body,
        out_shape=jax.ShapeDtypeStruct((N, H), jnp.float32),
    )(x, W1, od)

    p1 = _agg_kernel(hw1, ei_sc, zeros_feat).reshape(NC, N, H)

    hw2 = pl.pallas_call(
        _tc_mid_body,
        out_shape=jax.ShapeDtypeStruct((N, H), jnp.float32),
    )(p1, idg, od, b1.reshape(1, H), g1.reshape(1, H), be1.reshape(1, H), W2)

    p2 = _agg_kernel(hw2, ei_sc, zeros_feat).reshape(NC, N, H)

    out = pl.pallas_call(
        _tc_fin_body,
        out_shape=jax.ShapeDtypeStruct((1, C_OUT), jnp.float32),
    )(p2, idg, b2.reshape(1, H), g2.reshape(1, H), be2.reshape(1, H),
      Wc1, bc1.reshape(1, H), g3.reshape(1, H), be3.reshape(1, H),
      Wc3, bc3.reshape(1, C_OUT))
    return out


# speedup vs baseline: 1.1028x; 1.1028x over previous
import jax
import jax.numpy as jnp
from jax import lax
from jax.experimental import pallas as pl
from jax.experimental.pallas import tpu as pltpu
from jax.experimental.pallas import tpu_sc as plsc

N = 10000
E = 320000
H = 128
C_OUT = 10
NC = 2
NS = 16
NW = NC * NS
RPT = N // NS
T = 80
DT = 80
DCH = E // (NS * DT)
SCH = 128
EPW = SCH * T
NPAD = T
DEG_W = 16

_mesh = plsc.VectorSubcoreMesh(core_axis_name="c", subcore_axis_name="s")


DK = 10
DGRP = DCH // DK


def _deg_body(ei_ref, ones_ref, zeros_ref, out_ref, idx_v, ones_v, acc_sh,
              d0, d1, d2, d3, d4, d5, d6, d7, d8, d9):
    c = lax.axis_index("c")
    s = lax.axis_index("s")
    dsem = [d0, d1, d2, d3, d4, d5, d6, d7, d8, d9]

    @pl.when(s == 0)
    def _():
        pltpu.sync_copy(zeros_ref, acc_sh)

    pltpu.sync_copy(ei_ref.at[c, s], idx_v)
    pltpu.sync_copy(ones_ref, ones_v)
    plsc.subcore_barrier()

    def body(j, carry):
        descs = []
        for b in range(DK):
            descs.append(pltpu.async_copy(
                ones_v, acc_sh.at[idx_v.at[j * DK + b]], dsem[b], add=True))
        for d in descs:
            d.wait()
        return carry

    lax.fori_loop(0, DGRP, body, 0)
    plsc.subcore_barrier()

    @pl.when(s == 0)
    def _():
        pltpu.sync_copy(acc_sh, out_ref.at[c])


_deg_kernel = pl.kernel(
    _deg_body,
    out_type=jax.ShapeDtypeStruct((NC, N), jnp.float32),
    mesh=_mesh,
    scratch_types=[
        pltpu.VMEM((DCH, DT), jnp.int32),
        pltpu.VMEM((DT,), jnp.float32),
        pltpu.VMEM_SHARED((N,), jnp.float32),
        pltpu.SemaphoreType.DMA,
        pltpu.SemaphoreType.DMA,
        pltpu.SemaphoreType.DMA,
        pltpu.SemaphoreType.DMA,
        pltpu.SemaphoreType.DMA,
        pltpu.SemaphoreType.DMA,
        pltpu.SemaphoreType.DMA,
        pltpu.SemaphoreType.DMA,
        pltpu.SemaphoreType.DMA,
        pltpu.SemaphoreType.DMA,
    ],
    compiler_params=pltpu.CompilerParams(use_tc_tiling_on_sc=False),
)


NBUF = 4
SGRP = SCH // NBUF
SLOTS = 4
SGJ = SGRP // SLOTS


def _agg_body(hw_ref, ei_ref, zeros_ref, out_ref,
              ss0, ss1, ss2, ss3, ds0, ds1, ds2, ds3,
              r0, r1, r2, r3, acc_sh,
              l0, l1, l2, l3, g0, g1, g2, g3, s0, s1, s2, s3):
    c = lax.axis_index("c")
    s = lax.axis_index("s")
    wid = s * NC + c
    ss = [ss0, ss1, ss2, ss3]
    ds = [ds0, ds1, ds2, ds3]
    rows = [r0, r1, r2, r3]
    lsem = [l0, l1, l2, l3]
    gsem = [g0, g1, g2, g3]
    ssem = [s0, s1, s2, s3]

    def slab_src(g):
        return ei_ref.at[0, wid, pl.ds(g * NBUF, NBUF)]

    def slab_dst(g):
        return ei_ref.at[1, wid, pl.ds(g * NBUF, NBUF)]

    pltpu.sync_copy(zeros_ref, acc_sh.at[pl.ds(s * RPT, RPT)])
    pltpu.sync_copy(slab_src(0), ss[0])
    pltpu.sync_copy(slab_dst(0), ds[0])
    for k in range(1, SLOTS):
        pltpu.async_copy(slab_src(k), ss[k], lsem[k])
        pltpu.async_copy(slab_dst(k), ds[k], lsem[k])
    plsc.subcore_barrier()

    for b in range(NBUF):
        pltpu.async_copy(hw_ref.at[ss[0].at[b]], rows[b], gsem[b])

    def run_group(g, kk, knext, last):
        if not last:
            pltpu.make_async_copy(slab_src(g), ss[knext], lsem[knext]).wait()
            pltpu.make_async_copy(slab_dst(g), ds[knext], lsem[knext]).wait()
        descs = [None] * NBUF
        for b in range(NBUF):
            pltpu.make_async_copy(hw_ref.at[ss[kk].at[b]], rows[b],
                                  gsem[b]).wait()
            if b > 0:
                descs[b - 1].wait()
                if not last:
                    pltpu.async_copy(hw_ref.at[ss[knext].at[b - 1]],
                                     rows[b - 1], gsem[b - 1])
            descs[b] = pltpu.async_copy(rows[b], acc_sh.at[ds[kk].at[b]],
                                        ssem[b], add=True)
        descs[NBUF - 1].wait()
        if not last:
            pltpu.async_copy(hw_ref.at[ss[knext].at[NBUF - 1]],
                             rows[NBUF - 1], gsem[NBUF - 1])

    def body(j, carry):
        for gg in range(SLOTS):
            g = j * SLOTS + gg
            run_group(g, gg, (gg + 1) % SLOTS, last=False)
            pltpu.async_copy(slab_src(g + SLOTS), ss[gg], lsem[gg])
            pltpu.async_copy(slab_dst(g + SLOTS), ds[gg], lsem[gg])
        return carry

    lax.fori_loop(0, SGJ - 1, body, 0)

    for gg in range(SLOTS):
        g = (SGJ - 1) * SLOTS + gg
        run_group(g, gg, (gg + 1) % SLOTS, last=(gg == SLOTS - 1))

    plsc.subcore_barrier()
    pltpu.sync_copy(acc_sh.at[pl.ds(s * RPT, RPT)], out_ref.at[c, s])


_agg_kernel = pl.kernel(
    _agg_body,
    out_type=jax.ShapeDtypeStruct((NC, NS, RPT, H), jnp.float32),
    mesh=_mesh,
    scratch_types=[
        pltpu.VMEM((NBUF, T), jnp.int32),
        pltpu.VMEM((NBUF, T), jnp.int32),
        pltpu.VMEM((NBUF, T), jnp.int32),
        pltpu.VMEM((NBUF, T), jnp.int32),
        pltpu.VMEM((NBUF, T), jnp.int32),
        pltpu.VMEM((NBUF, T), jnp.int32),
        pltpu.VMEM((NBUF, T), jnp.int32),
        pltpu.VMEM((NBUF, T), jnp.int32),
        pltpu.VMEM((T, H), jnp.float32),
        pltpu.VMEM((T, H), jnp.float32),
        pltpu.VMEM((T, H), jnp.float32),
        pltpu.VMEM((T, H), jnp.float32),
        pltpu.VMEM_SHARED((N + NPAD, H), jnp.float32),
        pltpu.SemaphoreType.DMA,
        pltpu.SemaphoreType.DMA,
        pltpu.SemaphoreType.DMA,
        pltpu.SemaphoreType.DMA,
        pltpu.SemaphoreType.DMA,
        pltpu.SemaphoreType.DMA,
        pltpu.SemaphoreType.DMA,
        pltpu.SemaphoreType.DMA,
        pltpu.SemaphoreType.DMA,
        pltpu.SemaphoreType.DMA,
        pltpu.SemaphoreType.DMA,
        pltpu.SemaphoreType.DMA,
    ],
    compiler_params=pltpu.CompilerParams(use_tc_tiling_on_sc=False),
)


def _ln(x, g, b):
    mu = jnp.mean(x, axis=-1, keepdims=True)
    var = jnp.mean((x - mu) ** 2, axis=-1, keepdims=True)
    return (x - mu) * lax.rsqrt(var + 1e-5) * g + b


def _gelu(x):
    return 0.5 * x * (1.0 + lax.erf(x * 0.7071067811865476))


def _inv_sqrt_deg(d):
    return lax.rsqrt(jnp.maximum(d, 1.0))


def _tc_pre_body(x_ref, w_ref, od_ref, out_ref):
    h = x_ref[...] * _inv_sqrt_deg(od_ref[...])
    out_ref[...] = jnp.dot(h, w_ref[...], preferred_element_type=jnp.float32, precision=lax.Precision.HIGHEST)


def _tc_mid_body(p_ref, id_ref, od_ref, b1_ref, g1_ref, be1_ref, w2_ref, out_ref):
    agg = p_ref[0] + p_ref[1]
    agg = agg * _inv_sqrt_deg(id_ref[...]) + b1_ref[...]
    h = _gelu(_ln(agg, g1_ref[...], be1_ref[...]))
    h = h * _inv_sqrt_deg(od_ref[...])
    out_ref[...] = jnp.dot(h, w2_ref[...], preferred_element_type=jnp.float32, precision=lax.Precision.HIGHEST)


def _tc_fin_body(p_ref, id_ref, b2_ref, g2_ref, be2_ref, wc1_ref, bc1_ref,
                 g3_ref, be3_ref, wc3_ref, bc3_ref, out_ref):
    agg = p_ref[0] + p_ref[1]
    agg = agg * _inv_sqrt_deg(id_ref[...]) + b2_ref[...]
    h = _gelu(_ln(agg, g2_ref[...], be2_ref[...]))
    hg = jnp.mean(h, axis=0, keepdims=True)
    z = jnp.dot(hg, wc1_ref[...], preferred_element_type=jnp.float32, precision=lax.Precision.HIGHEST) + bc1_ref[...]
    z = jnp.maximum(_ln(z, g3_ref[...], be3_ref[...]), 0.0)
    out_ref[...] = jnp.dot(z, wc3_ref[...], preferred_element_type=jnp.float32, precision=lax.Precision.HIGHEST) + bc3_ref[...]


def kernel(x, edge_index, W1, b1, W2, b2, g1, be1, g2, be2, g3, be3,
           Wc1, bc1, Wc3, bc3):
    ei_deg = edge_index.reshape(2, NS, DCH, DT)
    pad = EPW - E // NW
    apad = jnp.arange(pad, dtype=jnp.int32)
    src_pad = jnp.broadcast_to((apad * 131) % N, (NW, pad))
    dst_pad = jnp.broadcast_to(N + (apad % NPAD), (NW, pad))
    ei_sc = jnp.concatenate(
        [edge_index.reshape(2, NW, E // NW),
         jnp.stack([src_pad, dst_pad])], axis=-1).reshape(2, NW, SCH, T)
    ones_deg = jnp.ones((DT,), jnp.float32)
    zeros_deg = jnp.zeros((N,), jnp.float32)
    zeros_feat = jnp.zeros((RPT, H), jnp.float32)

    deg = _deg_kernel(ei_deg, ones_deg, zeros_deg)
    od = deg[0].reshape(N, 1)
    idg = deg[1].reshape(N, 1)

    hw1 = pl.pallas_call(
        _tc_pre_body,
        out_shape=jax.ShapeDtypeStruct((N, H), jnp.float32),
    )(x, W1, od)

    p1 = _agg_kernel(hw1, ei_sc, zeros_feat).reshape(NC, N, H)

    hw2 = pl.pallas_call(
        _tc_mid_body,
        out_shape=jax.ShapeDtypeStruct((N, H), jnp.float32),
    )(p1, idg, od, b1.reshape(1, H), g1.reshape(1, H), be1.reshape(1, H), W2)

    p2 = _agg_kernel(hw2, ei_sc, zeros_feat).reshape(NC, N, H)

    out = pl.pallas_call(
        _tc_fin_body,
        out_shape=jax.ShapeDtypeStruct((1, C_OUT), jnp.float32),
    )(p2, idg, b2.reshape(1, H), g2.reshape(1, H), be2.reshape(1, H),
      Wc1, bc1.reshape(1, H), g3.reshape(1, H), be3.reshape(1, H),
      Wc3, bc3.reshape(1, C_OUT))
    return out
